# Initial kernel scaffold; baseline (speedup 1.0000x reference)
import jax
import jax.numpy as jnp
from jax.experimental import pallas as pl


def kernel(tokens, table, avg, var):
    raise NotImplementedError("write your pallas kernel here")


# speedup vs baseline: 4.4193x; 4.4193x over previous
import functools

import jax
import jax.numpy as jnp
import numpy as np
from jax import lax
from jax.experimental import pallas as pl
from jax.experimental.pallas import tpu as pltpu
from jax.experimental.pallas import tpu_sc as plsc

_VOCAB = 100000
_D = 64
_NTOK = 5
_BATCH = 4096
_SEQ = 200
_SEQ_E = _SEQ - _NTOK
_NW = 32
_BPW = _BATCH // _NW
_CHUNKS = ((0, 96), (96, 96), (192, 3))

@functools.partial(
    pl.kernel,
    out_type=jax.ShapeDtypeStruct((_BATCH, _SEQ, _D), jnp.float32),
    mesh=plsc.VectorSubcoreMesh(core_axis_name="c", subcore_axis_name="s"),
    compiler_params=pltpu.CompilerParams(use_tc_tiling_on_sc=False),
    scratch_types=[
        pltpu.VMEM((_BPW, _SEQ), jnp.int32),
        pltpu.VMEM((_SEQ_E, _D), jnp.float32),
        pltpu.VMEM((_NTOK, _D), jnp.float32),
        pltpu.VMEM((_NTOK, _D), jnp.float32),
        pltpu.VMEM((_NTOK, _D), jnp.float32),
        pltpu.VMEM((_NTOK, _D), jnp.float32),
        pltpu.SemaphoreType.DMA,
    ],
)
def _sc_embed(idx_hbm, table_hbm, var_hbm, avg_hbm, smp_hbm, out_hbm,
              idx_v, rows_v, var_v, avg_v, smp_v, pref_v, sem):
    nc = 2
    wid = lax.axis_index("s") * nc + lax.axis_index("c")
    b0 = wid * _BPW
    pltpu.sync_copy(idx_hbm.at[pl.ds(b0, _BPW)], idx_v)
    pltpu.sync_copy(var_hbm, var_v)
    pltpu.sync_copy(avg_hbm, avg_v)

    def step(j, carry):
        b = b0 + j
        for off, sz in _CHUNKS:
            pltpu.make_async_copy(
                table_hbm.at[idx_v.at[j, pl.ds(off, sz)]],
                rows_v.at[pl.ds(off, sz)],
                sem,
            ).start()
        pltpu.sync_copy(smp_hbm.at[b], smp_v)
        for r in range(_NTOK):
            for c in range(_D // 16):
                sl = pl.ds(c * 16, 16)
                pref_v[r, sl] = smp_v[r, sl] * var_v[r, sl] + avg_v[r, sl]
        pltpu.sync_copy(pref_v, out_hbm.at[b, pl.ds(_SEQ_E, _NTOK)])
        pltpu.make_async_copy(
            table_hbm.at[pl.ds(0, _SEQ_E)], rows_v, sem).wait()
        pltpu.sync_copy(rows_v, out_hbm.at[b, pl.ds(0, _SEQ_E)])
        return carry

    lax.fori_loop(0, _BPW, step, 0)


def kernel(tokens, table, avg, var):
    idx = jnp.roll(tokens, -_NTOK, axis=1)
    sample = jax.random.normal(
        jax.random.key(1), (_BATCH, _NTOK, _D), dtype=jnp.float32)
    return _sc_embed(idx, table, var, avg, sample)


# speedup vs baseline: 4.7079x; 1.0653x over previous
import functools

import jax
import jax.numpy as jnp
from jax import lax
from jax.experimental import pallas as pl
from jax.experimental.pallas import tpu as pltpu
from jax.experimental.pallas import tpu_sc as plsc

_VOCAB = 100000
_D = 64
_NTOK = 5
_BATCH = 4096
_SEQ = 200
_SEQ_E = _SEQ - _NTOK
_NW = 32
_BPW = _BATCH // _NW
_NBUF = 4
_HALF = _BPW // 2
_CHUNKS = ((0, 96), (96, 96), (192, 3))


@functools.partial(
    pl.kernel,
    out_type=jax.ShapeDtypeStruct((_BATCH, _SEQ, _D), jnp.float32),
    mesh=plsc.VectorSubcoreMesh(core_axis_name="c", subcore_axis_name="s"),
    compiler_params=pltpu.CompilerParams(use_tc_tiling_on_sc=False),
    scratch_types=[
        pltpu.VMEM((_BPW, _SEQ), jnp.int32),
        pltpu.VMEM((_SEQ_E, _D), jnp.float32),
        pltpu.VMEM((_SEQ_E, _D), jnp.float32),
        pltpu.VMEM((_SEQ_E, _D), jnp.float32),
        pltpu.VMEM((_SEQ_E, _D), jnp.float32),
        pltpu.VMEM((_NTOK, _D), jnp.float32),
        pltpu.VMEM((_NTOK, _D), jnp.float32),
        pltpu.VMEM((_NTOK, _D), jnp.float32),
        pltpu.VMEM((_NTOK, _D), jnp.float32),
        pltpu.SemaphoreType.DMA,
        pltpu.SemaphoreType.DMA,
        pltpu.SemaphoreType.DMA,
        pltpu.SemaphoreType.DMA,
        pltpu.SemaphoreType.DMA,
        pltpu.SemaphoreType.DMA,
        pltpu.SemaphoreType.DMA,
        pltpu.SemaphoreType.DMA,
    ],
)
def _sc_embed(idx_hbm, table_hbm, var_hbm, avg_hbm, smp_hbm, out_hbm,
              idx_v, r0, r1, r2, r3, var_v, avg_v, smp_v, pref_v,
              g0, g1, g2, g3, w0, w1, w2, w3):
    rows = (r0, r1, r2, r3)
    gsem = (g0, g1, g2, g3)
    wsem = (w0, w1, w2, w3)
    nc = 2
    wid = lax.axis_index("s") * nc + lax.axis_index("c")
    b0 = wid * _BPW
    pltpu.sync_copy(idx_hbm.at[pl.ds(b0, _BPW)], idx_v)
    pltpu.sync_copy(var_hbm, var_v)
    pltpu.sync_copy(avg_hbm, avg_v)

    def issue_gather(jloc, k):
        for off, sz in _CHUNKS:
            pltpu.make_async_copy(
                table_hbm.at[idx_v.at[jloc, pl.ds(off, sz)]],
                rows[k].at[pl.ds(off, sz)],
                gsem[k]).start()

    def drain_gather(k):
        pltpu.make_async_copy(
            table_hbm.at[pl.ds(0, _SEQ_E)], rows[k], gsem[k]).wait()

    def issue_write(jloc, k):
        pltpu.make_async_copy(
            rows[k], out_hbm.at[b0 + jloc, pl.ds(0, _SEQ_E)], wsem[k]).start()

    def drain_write(jloc, k):
        pltpu.make_async_copy(
            rows[k], out_hbm.at[b0 + jloc, pl.ds(0, _SEQ_E)], wsem[k]).wait()

    for k in range(_NBUF):
        issue_gather(k, k)

    def round_(j, carry):
        jb = j * _NBUF
        for k in range(_NBUF):
            b = b0 + jb + k
            pltpu.sync_copy(smp_hbm.at[b], smp_v)
            for r in range(_NTOK):
                for c in range(_D // 16):
                    sl = pl.ds(c * 16, 16)
                    pref_v[r, sl] = (
                        smp_v[r, sl] * var_v[r, sl] + avg_v[r, sl])
            pltpu.sync_copy(pref_v, out_hbm.at[b, pl.ds(_SEQ_E, _NTOK)])
            drain_gather(k)
            issue_write(jb + k, k)
        for k in range(_NBUF):
            nxt = jb + _NBUF + k

            @pl.when(nxt < _BPW)
            def _():
                drain_write(nxt - _NBUF, k)
                issue_gather(nxt, k)

        return carry

    lax.fori_loop(0, _BPW // _NBUF, round_, 0)
    for k in range(_NBUF):
        drain_write(_BPW - _NBUF + k, k)


def kernel(tokens, table, avg, var):
    idx = jnp.roll(tokens, -_NTOK, axis=1)
    sample = jax.random.normal(
        jax.random.key(1), (_BATCH, _NTOK, _D), dtype=jnp.float32)
    return _sc_embed(idx, table, var, avg, sample)


# speedup vs baseline: 4.7793x; 1.0152x over previous
import functools

import jax
import jax.numpy as jnp
from jax import lax
from jax.experimental import pallas as pl
from jax.experimental.pallas import tpu as pltpu
from jax.experimental.pallas import tpu_sc as plsc

_VOCAB = 100000
_D = 64
_NTOK = 5
_BATCH = 4096
_SEQ = 200
_SEQ_E = _SEQ - _NTOK
_NW = 32
_BPW = _BATCH // _NW
_NBUF = 4
_HALF = _BPW // 2
_CHUNKS = ((0, 96), (96, 96), (192, 3))


@functools.partial(
    pl.kernel,
    out_type=jax.ShapeDtypeStruct((_BATCH, _SEQ, _D), jnp.float32),
    mesh=plsc.VectorSubcoreMesh(core_axis_name="c", subcore_axis_name="s"),
    compiler_params=pltpu.CompilerParams(use_tc_tiling_on_sc=False),
    scratch_types=[
        pltpu.VMEM((_BPW, _SEQ), jnp.int32),
        pltpu.VMEM((_SEQ_E, _D), jnp.float32),
        pltpu.VMEM((_SEQ_E, _D), jnp.float32),
        pltpu.VMEM((_SEQ_E, _D), jnp.float32),
        pltpu.VMEM((_SEQ_E, _D), jnp.float32),
        pltpu.VMEM((_NTOK, _D), jnp.float32),
        pltpu.VMEM((_NTOK, _D), jnp.float32),
        pltpu.VMEM((_BPW, _NTOK, _D), jnp.float32),
        pltpu.SemaphoreType.DMA,
        pltpu.SemaphoreType.DMA,
        pltpu.SemaphoreType.DMA,
        pltpu.SemaphoreType.DMA,
        pltpu.SemaphoreType.DMA,
        pltpu.SemaphoreType.DMA,
        pltpu.SemaphoreType.DMA,
        pltpu.SemaphoreType.DMA,
        pltpu.SemaphoreType.DMA,
    ],
)
def _sc_embed(idx_hbm, table_hbm, var_hbm, avg_hbm, smp_hbm, out_hbm,
              idx_v, r0, r1, r2, r3, var_v, avg_v, smp_v,
              g0, g1, g2, g3, w0, w1, w2, w3, psem):
    rows = (r0, r1, r2, r3)
    gsem = (g0, g1, g2, g3)
    wsem = (w0, w1, w2, w3)
    nc = 2
    wid = lax.axis_index("s") * nc + lax.axis_index("c")
    b0 = wid * _BPW
    pltpu.sync_copy(idx_hbm.at[pl.ds(b0, _BPW)], idx_v)
    pltpu.sync_copy(var_hbm, var_v)
    pltpu.sync_copy(avg_hbm, avg_v)

    def issue_gather(jloc, k):
        for off, sz in _CHUNKS:
            pltpu.make_async_copy(
                table_hbm.at[idx_v.at[jloc, pl.ds(off, sz)]],
                rows[k].at[pl.ds(off, sz)],
                gsem[k]).start()

    def drain_gather(k):
        pltpu.make_async_copy(
            table_hbm.at[pl.ds(0, _SEQ_E)], rows[k], gsem[k]).wait()

    def issue_write(jloc, k):
        pltpu.make_async_copy(
            rows[k], out_hbm.at[b0 + jloc, pl.ds(0, _SEQ_E)], wsem[k]).start()

    def drain_write(jloc, k):
        pltpu.make_async_copy(
            rows[k], out_hbm.at[b0 + jloc, pl.ds(0, _SEQ_E)], wsem[k]).wait()

    for k in range(_NBUF):
        issue_gather(k, k)

    pltpu.sync_copy(smp_hbm.at[pl.ds(b0, _BPW)], smp_v)

    def fma(m, carry):
        for r in range(_NTOK):
            for c in range(_D // 16):
                sl = pl.ds(c * 16, 16)
                smp_v[m, r, sl] = (
                    smp_v[m, r, sl] * var_v[r, sl] + avg_v[r, sl])
        return carry

    lax.fori_loop(0, _BPW, fma, 0)

    def issue_pref(jloc):
        pltpu.make_async_copy(
            smp_v.at[jloc], out_hbm.at[b0 + jloc, pl.ds(_SEQ_E, _NTOK)],
            psem).start()

    def round_(j, carry):
        jb = j * _NBUF
        for k in range(_NBUF):
            issue_pref(jb + k)
            drain_gather(k)
            issue_write(jb + k, k)
        for k in range(_NBUF):
            nxt = jb + _NBUF + k

            @pl.when(nxt < _BPW)
            def _():
                drain_write(nxt - _NBUF, k)
                issue_gather(nxt, k)

        return carry

    lax.fori_loop(0, _BPW // _NBUF, round_, 0)
    for k in range(_NBUF):
        drain_write(_BPW - _NBUF + k, k)

    def drain_pref(i, carry):
        pltpu.make_async_copy(
            smp_v.at[0], out_hbm.at[b0, pl.ds(_SEQ_E, _NTOK)], psem).wait()
        return carry

    lax.fori_loop(0, _BPW, drain_pref, 0)


def kernel(tokens, table, avg, var):
    idx = jnp.roll(tokens, -_NTOK, axis=1)
    sample = jax.random.normal(
        jax.random.key(1), (_BATCH, _NTOK, _D), dtype=jnp.float32)
    return _sc_embed(idx, table, var, avg, sample)


# speedup vs baseline: 4.7816x; 1.0005x over previous
import functools

import jax
import jax.numpy as jnp
from jax import lax
from jax.experimental import pallas as pl
from jax.experimental.pallas import tpu as pltpu
from jax.experimental.pallas import tpu_sc as plsc

_VOCAB = 100000
_D = 64
_NTOK = 5
_BATCH = 4096
_SEQ = 200
_SEQ_E = _SEQ - _NTOK
_NW = 32
_BPW = _BATCH // _NW
_NBUF = 4
_HALF = _BPW // 2
_CHUNKS = ((0, 192), (192, 3))


@functools.partial(
    pl.kernel,
    out_type=jax.ShapeDtypeStruct((_BATCH, _SEQ, _D), jnp.float32),
    mesh=plsc.VectorSubcoreMesh(core_axis_name="c", subcore_axis_name="s"),
    compiler_params=pltpu.CompilerParams(use_tc_tiling_on_sc=False),
    scratch_types=[
        pltpu.VMEM((_BPW, _SEQ), jnp.int32),
        pltpu.VMEM((_SEQ_E, _D), jnp.float32),
        pltpu.VMEM((_SEQ_E, _D), jnp.float32),
        pltpu.VMEM((_SEQ_E, _D), jnp.float32),
        pltpu.VMEM((_SEQ_E, _D), jnp.float32),
        pltpu.VMEM((_NTOK, _D), jnp.float32),
        pltpu.VMEM((_NTOK, _D), jnp.float32),
        pltpu.VMEM((_BPW, _NTOK, _D), jnp.float32),
        pltpu.SemaphoreType.DMA,
        pltpu.SemaphoreType.DMA,
        pltpu.SemaphoreType.DMA,
        pltpu.SemaphoreType.DMA,
        pltpu.SemaphoreType.DMA,
        pltpu.SemaphoreType.DMA,
        pltpu.SemaphoreType.DMA,
        pltpu.SemaphoreType.DMA,
        pltpu.SemaphoreType.DMA,
    ],
)
def _sc_embed(idx_hbm, table_hbm, var_hbm, avg_hbm, smp_hbm, out_hbm,
              idx_v, r0, r1, r2, r3, var_v, avg_v, smp_v,
              g0, g1, g2, g3, w0, w1, w2, w3, psem):
    rows = (r0, r1, r2, r3)
    gsem = (g0, g1, g2, g3)
    wsem = (w0, w1, w2, w3)
    nc = 2
    wid = lax.axis_index("s") * nc + lax.axis_index("c")
    b0 = wid * _BPW
    pltpu.sync_copy(idx_hbm.at[pl.ds(b0, _BPW)], idx_v)
    pltpu.sync_copy(var_hbm, var_v)
    pltpu.sync_copy(avg_hbm, avg_v)

    def issue_gather(jloc, k):
        for off, sz in _CHUNKS:
            pltpu.make_async_copy(
                table_hbm.at[idx_v.at[jloc, pl.ds(off, sz)]],
                rows[k].at[pl.ds(off, sz)],
                gsem[k]).start()

    def drain_gather(k):
        pltpu.make_async_copy(
            table_hbm.at[pl.ds(0, _SEQ_E)], rows[k], gsem[k]).wait()

    def issue_write(jloc, k):
        pltpu.make_async_copy(
            rows[k], out_hbm.at[b0 + jloc, pl.ds(0, _SEQ_E)], wsem[k]).start()

    def drain_write(jloc, k):
        pltpu.make_async_copy(
            rows[k], out_hbm.at[b0 + jloc, pl.ds(0, _SEQ_E)], wsem[k]).wait()

    for k in range(_NBUF):
        issue_gather(k, k)

    pltpu.sync_copy(smp_hbm.at[pl.ds(b0, _BPW)], smp_v)

    def fma(m, carry):
        for r in range(_NTOK):
            for c in range(_D // 16):
                sl = pl.ds(c * 16, 16)
                smp_v[m, r, sl] = (
                    smp_v[m, r, sl] * var_v[r, sl] + avg_v[r, sl])
        return carry

    lax.fori_loop(0, _BPW, fma, 0)

    def issue_pref(jloc):
        pltpu.make_async_copy(
            smp_v.at[jloc], out_hbm.at[b0 + jloc, pl.ds(_SEQ_E, _NTOK)],
            psem).start()

    def round_(j, carry):
        jb = j * _NBUF
        for k in range(_NBUF):
            issue_pref(jb + k)
            drain_gather(k)
            issue_write(jb + k, k)
        for k in range(_NBUF):
            nxt = jb + _NBUF + k

            @pl.when(nxt < _BPW)
            def _():
                drain_write(nxt - _NBUF, k)
                issue_gather(nxt, k)

        return carry

    lax.fori_loop(0, _BPW // _NBUF, round_, 0)
    for k in range(_NBUF):
        drain_write(_BPW - _NBUF + k, k)

    def drain_pref(i, carry):
        pltpu.make_async_copy(
            smp_v.at[0], out_hbm.at[b0, pl.ds(_SEQ_E, _NTOK)], psem).wait()
        return carry

    lax.fori_loop(0, _BPW, drain_pref, 0)


def kernel(tokens, table, avg, var):
    idx = jnp.roll(tokens, -_NTOK, axis=1)
    sample = jax.random.normal(
        jax.random.key(1), (_BATCH, _NTOK, _D), dtype=jnp.float32)
    return _sc_embed(idx, table, var, avg, sample)
